# baseline (speedup 1.0000x reference)
import functools

import jax
import jax.numpy as jnp
from jax.experimental import pallas as pl

GROUPS = [
    [0, 1], [1, 2, 3, 4, 5], [2, 3, 6], [3, 6, 7], [6, 7], [2, 4, 8],
    [4, 8, 9], [8, 9], [10, 11, 12], [11, 12, 13], [12, 13], [10, 14, 15],
    [14, 15, 16], [15, 16], [5, 10, 11, 14], [2, 5, 10], [0, 1, 2],
]

_B, _J, _C, _D = 16384, 17, 3, 32
_O = 3 * len(GROUPS)


def _build_wexp(weights, biases):
    wexp = jnp.zeros((_J * _C, _O), dtype=jnp.float32)
    bias_sum = jnp.zeros((_O,), dtype=jnp.float32)
    for i, (g, w, b) in enumerate(zip(GROUPS, weights, biases)):
        for k, j in enumerate(g):
            for c in range(_C):
                wexp = wexp.at[j * _C + c, 3 * i + c].add(w[k, c])
        bias_sum = bias_sum.at[3 * i : 3 * i + 3].add(jnp.sum(b, axis=0))
    wexp_full = jnp.repeat(wexp, _D, axis=0)
    return wexp_full, bias_sum


def _mm_body(x_ref, w_ref, b_ref, o_ref):
    o_ref[...] = (
        jnp.dot(x_ref[...], w_ref[...], preferred_element_type=jnp.float32)
        + b_ref[...]
    )


@functools.partial(jax.jit, static_argnames=())
def _run(x2d, wexp, bias_row):
    bb = 2048
    grid = (_B // bb,)
    return pl.pallas_call(
        _mm_body,
        grid=grid,
        in_specs=[
            pl.BlockSpec((bb, _J * _C * _D), lambda i: (i, 0)),
            pl.BlockSpec((_J * _C * _D, _O), lambda i: (0, 0)),
            pl.BlockSpec((1, _O), lambda i: (0, 0)),
        ],
        out_specs=pl.BlockSpec((bb, _O), lambda i: (i, 0)),
        out_shape=jax.ShapeDtypeStruct((_B, _O), jnp.float32),
    )(x2d, wexp, bias_row)


def kernel(input, weights, biases):
    wexp, bias_sum = _build_wexp(weights, biases)
    x2d = input.reshape(_B, _J * _C * _D)
    return _run(x2d, wexp, bias_sum[None, :])
